# baseline (speedup 1.0000x reference)
import jax
import jax.numpy as jnp
from jax import lax
from jax.experimental import pallas as pl
from jax.experimental.pallas import tpu as pltpu
from jax.experimental.pallas import tpu_sc as plsc

N_GENES = 1000000
N_DRUGS = 1000
EMB_DIM = 64
BATCH = 16384
HDIM = EMB_DIM // 2

NC = 2
NS = 16
LANES = 16
NW = NC * NS
B_PER_W = BATCH // NW
IDX_CHUNK = 128
N_CHUNKS = B_PER_W // IDX_CHUNK
D_VREGS = EMB_DIM // LANES


def _body(h1_hbm, h2_hbm, drug_hbm, gi_hbm, di_hbm, w_hbm, out_hbm,
          gidx_v, didx_v, growsA, growsB, drows, w_v, pscr, out_v, sem):
    wid = lax.axis_index("s") * NC + lax.axis_index("c")
    base = wid * B_PER_W

    pltpu.sync_copy(gi_hbm.at[wid], gidx_v)
    pltpu.sync_copy(di_hbm.at[wid], didx_v)
    pltpu.sync_copy(w_hbm, w_v)

    copies = []
    for c in range(N_CHUNKS):
        copies.append(pltpu.async_copy(
            h1_hbm.at[gidx_v.at[c]],
            growsA.at[pl.ds(c * IDX_CHUNK, IDX_CHUNK)], sem))
        copies.append(pltpu.async_copy(
            h2_hbm.at[gidx_v.at[c]],
            growsB.at[pl.ds(c * IDX_CHUNK, IDX_CHUNK)], sem))
        copies.append(pltpu.async_copy(
            drug_hbm.at[didx_v.at[c]],
            drows.at[pl.ds(c * IDX_CHUNK, IDX_CHUNK)], sem))
    for cp in copies:
        cp.wait()

    wregs = [w_v[pl.ds(c * LANES, LANES)] for c in range(D_VREGS)]
    iota = lax.broadcasted_iota(jnp.int32, (LANES,), 0)
    colbase = iota * LANES

    def group_body(g, carry):
        rowbase = g * LANES
        for r in range(LANES):
            row = rowbase + r
            acc = None
            for c in range(D_VREGS):
                if c < 2:
                    gv = growsA[row, pl.ds(c * LANES, LANES)]
                else:
                    gv = growsB[row, pl.ds((c - 2) * LANES, LANES)]
                dv = drows[row, pl.ds(c * LANES, LANES)]
                t = gv * dv * wregs[c]
                acc = t if acc is None else acc + t
            pscr[pl.ds(r * LANES, LANES)] = acc
        cv = colbase
        tot = plsc.load_gather(pscr, [cv])
        for _ in range(LANES - 1):
            cv = cv + 1
            tot = tot + plsc.load_gather(pscr, [cv])
        out_v[pl.ds(rowbase, LANES)] = tot
        return carry

    lax.fori_loop(0, B_PER_W // LANES, group_body, 0)
    pltpu.sync_copy(out_v, out_hbm.at[pl.ds(base, B_PER_W)])


def _dist_mult_sc(h1, h2, drug_emb, gi, di, w):
    mesh = plsc.VectorSubcoreMesh(core_axis_name="c", subcore_axis_name="s",
                                  num_cores=NC, num_subcores=NS)
    return pl.kernel(
        _body,
        out_type=jax.ShapeDtypeStruct((BATCH,), jnp.float32),
        mesh=mesh,
        compiler_params=pltpu.CompilerParams(needs_layout_passes=False,
                                             use_tc_tiling_on_sc=False),
        scratch_types=[
            pltpu.VMEM((N_CHUNKS, IDX_CHUNK), jnp.int32),
            pltpu.VMEM((N_CHUNKS, IDX_CHUNK), jnp.int32),
            pltpu.VMEM((B_PER_W, HDIM), jnp.float32),
            pltpu.VMEM((B_PER_W, HDIM), jnp.float32),
            pltpu.VMEM((B_PER_W, EMB_DIM), jnp.float32),
            pltpu.VMEM((EMB_DIM,), jnp.float32),
            pltpu.VMEM((LANES * LANES,), jnp.float32),
            pltpu.VMEM((B_PER_W,), jnp.float32),
            pltpu.SemaphoreType.DMA,
        ],
    )(h1, h2, drug_emb, gi, di, w)


def kernel(gene_idx, drug_idx, gene_emb, drug_emb, W):
    h1 = gene_emb[:, :HDIM]
    h2 = gene_emb[:, HDIM:]
    gi = gene_idx.astype(jnp.int32).reshape(NW, N_CHUNKS, IDX_CHUNK)
    di = drug_idx.astype(jnp.int32).reshape(NW, N_CHUNKS, IDX_CHUNK)
    return _dist_mult_sc(h1, h2, drug_emb, gi, di, W)


# speedup vs baseline: 3.7330x; 3.7330x over previous
import jax
import jax.numpy as jnp
from jax import lax
from jax.experimental import pallas as pl
from jax.experimental.pallas import tpu as pltpu
from jax.experimental.pallas import tpu_sc as plsc

N_GENES = 1000000
N_DRUGS = 1000
EMB_DIM = 64
BATCH = 16384

NC = 2
NS = 16
LANES = 16
NW = NC * NS
B_PER_W = BATCH // NW
N_GROUPS = B_PER_W // LANES


def _body(gt_hbm, dt_hbm, gi_hbm, di_hbm, w_hbm, out_hbm,
          gidx_v, didx_v, slabs_v, dtab_v, w_v, out_v, sem):
    wid = lax.axis_index("s") * NC + lax.axis_index("c")
    base = wid * B_PER_W

    pltpu.sync_copy(gi_hbm.at[wid], gidx_v)
    pltpu.sync_copy(di_hbm.at[wid], didx_v)
    pltpu.sync_copy(w_hbm, w_v)
    pltpu.sync_copy(dt_hbm, dtab_v)

    wscal = []
    for c in range(EMB_DIM // LANES):
        wv = w_v[pl.ds(c * LANES, LANES)]
        for i in range(LANES):
            wscal.append(wv[i])

    iota = lax.broadcasted_iota(jnp.int32, (LANES,), 0)
    tconst = [jnp.full((LANES,), t, jnp.int32) for t in range(8)]
    sconst = [jnp.full((LANES,), s, jnp.int32) for s in range(8)]

    def group_body(k, carry):
        roff = k * LANES
        gvec = gidx_v[pl.ds(roff, LANES)]
        dvec = didx_v[pl.ds(roff, LANES)]
        for i in range(LANES):
            g = gvec[i]
            gtile = pl.multiple_of(g & ~127, 128)
            ph = (g >> 4) & 7
            tile_ref = gt_hbm.at[:, :, pl.ds(gtile, 128)]
            for P in range(8):
                @pl.when(ph == P)
                def _(tile_ref=tile_ref, P=P, i=i):
                    pltpu.async_copy(
                        tile_ref.at[:, :, pl.ds(P * LANES, LANES)],
                        slabs_v.at[:, :, pl.ds(i * LANES, LANES)], sem)
        for i in range(LANES):
            pltpu.make_async_copy(
                gt_hbm.at[:, :, pl.ds(0, LANES)],
                slabs_v.at[:, :, pl.ds(i * LANES, LANES)], sem).wait()
        gsel = iota * LANES + (gvec & 15)
        acc = jnp.zeros((LANES,), jnp.float32)
        for t in range(8):
            for s in range(8):
                gv = plsc.load_gather(slabs_v, [tconst[t], sconst[s], gsel])
                dv = plsc.load_gather(dtab_v, [tconst[t], sconst[s], dvec])
                acc = acc + gv * dv * wscal[t * 8 + s]
        out_v[pl.ds(roff, LANES)] = acc
        return carry

    lax.fori_loop(0, N_GROUPS, group_body, 0)
    pltpu.sync_copy(out_v, out_hbm.at[pl.ds(base, B_PER_W)])


def _dist_mult_sc(gt3, dt3, gi, di, w):
    mesh = plsc.VectorSubcoreMesh(core_axis_name="c", subcore_axis_name="s",
                                  num_cores=NC, num_subcores=NS)
    return pl.kernel(
        _body,
        out_type=jax.ShapeDtypeStruct((BATCH,), jnp.float32),
        mesh=mesh,
        compiler_params=pltpu.CompilerParams(needs_layout_passes=False,
                                             use_tc_tiling_on_sc=True),
        scratch_types=[
            pltpu.VMEM((B_PER_W,), jnp.int32),
            pltpu.VMEM((B_PER_W,), jnp.int32),
            pltpu.VMEM((8, 8, LANES * LANES), jnp.float32),
            pltpu.VMEM((8, 8, N_DRUGS), jnp.float32),
            pltpu.VMEM((EMB_DIM,), jnp.float32),
            pltpu.VMEM((B_PER_W,), jnp.float32),
            pltpu.SemaphoreType.DMA,
        ],
    )(gt3, dt3, gi, di, w)


def kernel(gene_idx, drug_idx, gene_emb, drug_emb, W):
    gt3 = gene_emb.T.reshape(8, 8, N_GENES)
    dt3 = drug_emb.T.reshape(8, 8, N_DRUGS)
    gi = gene_idx.astype(jnp.int32).reshape(NW, B_PER_W)
    di = drug_idx.astype(jnp.int32).reshape(NW, B_PER_W)
    return _dist_mult_sc(gt3, dt3, gi, di, W)


# speedup vs baseline: 3.7994x; 1.0178x over previous
import jax
import jax.numpy as jnp
from jax import lax
from jax.experimental import pallas as pl
from jax.experimental.pallas import tpu as pltpu
from jax.experimental.pallas import tpu_sc as plsc

N_GENES = 1000000
N_DRUGS = 1000
EMB_DIM = 64
BATCH = 16384

NC = 2
NS = 16
LANES = 16
NW = NC * NS
B_PER_W = BATCH // NW
N_GROUPS = B_PER_W // LANES


def _body(gt_hbm, dt_hbm, gi_hbm, di_hbm, w_hbm, out_hbm,
          gidx_v, didx_v, slabs_v, dtab_v, w_v, out_v, sem):
    wid = lax.axis_index("s") * NC + lax.axis_index("c")
    base = wid * B_PER_W

    pltpu.sync_copy(gi_hbm.at[wid], gidx_v)
    pltpu.sync_copy(di_hbm.at[wid], didx_v)
    pltpu.sync_copy(w_hbm, w_v)
    pltpu.sync_copy(dt_hbm, dtab_v)

    wscal = []
    for c in range(EMB_DIM // LANES):
        wv = w_v[pl.ds(c * LANES, LANES)]
        for i in range(LANES):
            wscal.append(wv[i])

    iota = lax.broadcasted_iota(jnp.int32, (LANES,), 0)
    tconst = [jnp.full((LANES,), t, jnp.int32) for t in range(8)]
    sconst = [jnp.full((LANES,), s, jnp.int32) for s in range(8)]

    HALFW = LANES * LANES

    def fire_group(k):
        roff = k * LANES
        gvec = gidx_v[pl.ds(roff, LANES)]
        par = (k & 1) * HALFW
        half_ref = slabs_v.at[:, :, pl.ds(pl.multiple_of(par, 128), HALFW)]
        for i in range(LANES):
            g = gvec[i]
            gtile = pl.multiple_of(g & ~127, 128)
            ph = (g >> 4) & 7
            tile_ref = gt_hbm.at[:, :, pl.ds(gtile, 128)]
            for P in range(8):
                @pl.when(ph == P)
                def _(tile_ref=tile_ref, P=P, i=i, half_ref=half_ref):
                    pltpu.async_copy(
                        tile_ref.at[:, :, pl.ds(P * LANES, LANES)],
                        half_ref.at[:, :, pl.ds(i * LANES, LANES)], sem)

    def drain_group():
        for i in range(LANES):
            pltpu.make_async_copy(
                gt_hbm.at[:, :, pl.ds(0, LANES)],
                slabs_v.at[:, :, pl.ds(i * LANES, LANES)], sem).wait()

    def compute_group(k):
        roff = k * LANES
        gvec = gidx_v[pl.ds(roff, LANES)]
        dvec = didx_v[pl.ds(roff, LANES)]
        gsel = (k & 1) * HALFW + iota * LANES + (gvec & 15)
        acc = jnp.zeros((LANES,), jnp.float32)
        for t in range(8):
            for s in range(8):
                gv = plsc.load_gather(slabs_v, [tconst[t], sconst[s], gsel])
                dv = plsc.load_gather(dtab_v, [tconst[t], sconst[s], dvec])
                acc = acc + gv * dv * wscal[t * 8 + s]
        out_v[pl.ds(roff, LANES)] = acc

    def group_body(k, carry):
        fire_group(k)

        @pl.when(k > 0)
        def _():
            compute_group(k - 1)

        drain_group()
        return carry

    lax.fori_loop(0, N_GROUPS, group_body, 0)
    compute_group(N_GROUPS - 1)
    pltpu.sync_copy(out_v, out_hbm.at[pl.ds(base, B_PER_W)])


def _dist_mult_sc(gt3, dt3, gi, di, w):
    mesh = plsc.VectorSubcoreMesh(core_axis_name="c", subcore_axis_name="s",
                                  num_cores=NC, num_subcores=NS)
    return pl.kernel(
        _body,
        out_type=jax.ShapeDtypeStruct((BATCH,), jnp.float32),
        mesh=mesh,
        compiler_params=pltpu.CompilerParams(needs_layout_passes=False,
                                             use_tc_tiling_on_sc=True),
        scratch_types=[
            pltpu.VMEM((B_PER_W,), jnp.int32),
            pltpu.VMEM((B_PER_W,), jnp.int32),
            pltpu.VMEM((8, 8, 2 * LANES * LANES), jnp.float32),
            pltpu.VMEM((8, 8, N_DRUGS), jnp.float32),
            pltpu.VMEM((EMB_DIM,), jnp.float32),
            pltpu.VMEM((B_PER_W,), jnp.float32),
            pltpu.SemaphoreType.DMA,
        ],
    )(gt3, dt3, gi, di, w)


def kernel(gene_idx, drug_idx, gene_emb, drug_emb, W):
    gt3 = gene_emb.T.reshape(8, 8, N_GENES)
    dt3 = drug_emb.T.reshape(8, 8, N_DRUGS)
    gi = gene_idx.astype(jnp.int32).reshape(NW, B_PER_W)
    di = drug_idx.astype(jnp.int32).reshape(NW, B_PER_W)
    return _dist_mult_sc(gt3, dt3, gi, di, W)
